# baseline (speedup 1.0000x reference)
import jax
import jax.numpy as jnp
from jax.experimental import pallas as pl
from jax.experimental.pallas import tpu as pltpu

_WIN = 7
_NP = _WIN * _WIN
_COV_NORM = _NP / (_NP - 1)
_K1, _K2 = 0.01, 0.03

_H = 512
_W = 512
_HO = _H - (_WIN - 1)
_WO = _W - (_WIN - 1)
_STRIP = 256
_NSTRIP = _HO // _STRIP
_REM = _HO - _NSTRIP * _STRIP


def _win7_rows(x):
    w2 = x[0:-1] + x[1:]
    w4 = w2[0:-2] + w2[2:]
    w6 = w4[0:-2] + w2[4:]
    return w6[0:-1] + x[6:]


def _ssim_map(p, t, band, c1s, c2s, out_rows):
    rs = jnp.concatenate([
        _win7_rows(p)[:out_rows],
        _win7_rows(t)[:out_rows],
        _win7_rows(p * p)[:out_rows],
        _win7_rows(t * t)[:out_rows],
        _win7_rows(p * t)[:out_rows],
    ], axis=0).astype(jnp.bfloat16)
    sums = jnp.dot(rs, band, preferred_element_type=jnp.float32)
    e = sums[0 * out_rows:1 * out_rows]
    f = sums[1 * out_rows:2 * out_rows]
    g = sums[2 * out_rows:3 * out_rows]
    h = sums[3 * out_rows:4 * out_rows]
    k = sums[4 * out_rows:5 * out_rows]
    ef2 = 2.0 * (e * f)
    sq = e * e + f * f
    a1 = ef2 + c1s
    b1 = sq + c1s
    a2 = (2.0 * _NP) * k - ef2 + c2s
    b2 = _NP * (g + h) - sq + c2s
    s = (a1 * a2) / (b1 * b2)
    lane = jax.lax.broadcasted_iota(jnp.int32, s.shape, 1)
    return jnp.where(lane < _WO, s, 0.0)


def _ssim_kernel(p_ref, t_ref, b_ref, o_ref):
    C = p_ref.shape[1]

    p_all = p_ref[...]
    dr = jnp.max(p_all) - jnp.min(p_all)
    c1s = (_K1 * dr) ** 2 * (_NP * _NP)
    c2s = (_K2 * dr) ** 2 * (_NP * (_NP - 1))
    band = b_ref[...]

    def body(idx, acc):
        c = idx // _NSTRIP
        i = idx - c * _NSTRIP
        r0 = i * _STRIP
        p = p_ref[0, c, pl.ds(r0, _STRIP + _WIN - 1), :]
        t = t_ref[0, c, pl.ds(r0, _STRIP + _WIN - 1), :]
        return acc + _ssim_map(p, t, band, c1s, c2s, _STRIP)

    acc = jnp.zeros((_STRIP, _W), jnp.float32)
    acc = jax.lax.fori_loop(0, C * _NSTRIP, body, acc)

    def tail(c, acc):
        r0 = _NSTRIP * _STRIP
        p = p_ref[0, c, pl.ds(r0, _H - r0), :]
        t = t_ref[0, c, pl.ds(r0, _H - r0), :]
        s = _ssim_map(p, t, band, c1s, c2s, _REM)
        pad = jnp.zeros((_STRIP - _REM, _W), jnp.float32)
        return acc + jnp.concatenate([s, pad], axis=0)

    acc = jax.lax.fori_loop(0, C, tail, acc)

    mean_s = jnp.sum(acc) * (1.0 / (C * _HO * _WO))
    o_ref[...] = jnp.full(o_ref.shape, mean_s, jnp.float32)


def kernel(pred, target):
    B, C, H, W = pred.shape
    row = jax.lax.broadcasted_iota(jnp.int32, (W, W), 0)
    col = jax.lax.broadcasted_iota(jnp.int32, (W, W), 1)
    d = row - col
    band = jnp.where((d >= 0) & (d <= _WIN - 1) & (col < _WO), 1.0, 0.0)
    band = band.astype(jnp.bfloat16)
    per_sample = pl.pallas_call(
        _ssim_kernel,
        grid=(B,),
        in_specs=[
            pl.BlockSpec((1, C, H, W), lambda b: (b, 0, 0, 0)),
            pl.BlockSpec((1, C, H, W), lambda b: (b, 0, 0, 0)),
            pl.BlockSpec((W, W), lambda b: (0, 0)),
        ],
        out_specs=pl.BlockSpec((1, 1, 128), lambda b: (b, 0, 0)),
        out_shape=jax.ShapeDtypeStruct((B, 1, 128), jnp.float32),
        compiler_params=pltpu.CompilerParams(
            dimension_semantics=("parallel",),
            vmem_limit_bytes=56 * 1024 * 1024,
        ),
    )(pred, target, band)
    return 1.0 - jnp.mean(per_sample[:, 0, 0])


# speedup vs baseline: 1.0384x; 1.0384x over previous
import jax
import jax.numpy as jnp
from jax.experimental import pallas as pl
from jax.experimental.pallas import tpu as pltpu

_WIN = 7
_NP = _WIN * _WIN
_COV_NORM = _NP / (_NP - 1)
_K1, _K2 = 0.01, 0.03

_H = 512
_W = 512
_HO = _H - (_WIN - 1)
_WO = _W - (_WIN - 1)
_STRIP = 128
_NSTRIP = _HO // _STRIP
_REM = _HO - _NSTRIP * _STRIP


def _win7_rows(x):
    w2 = x[0:-1] + x[1:]
    w4 = w2[0:-2] + w2[2:]
    w6 = w4[0:-2] + w2[4:]
    return w6[0:-1] + x[6:]


def _ssim_map(p, t, band, c1s, c2s, out_rows):
    rs = jnp.concatenate([
        _win7_rows(p)[:out_rows],
        _win7_rows(t)[:out_rows],
        _win7_rows(p * p)[:out_rows],
        _win7_rows(t * t)[:out_rows],
        _win7_rows(p * t)[:out_rows],
    ], axis=0).astype(jnp.bfloat16)
    sums = jnp.dot(rs, band, preferred_element_type=jnp.float32)
    e = sums[0 * out_rows:1 * out_rows]
    f = sums[1 * out_rows:2 * out_rows]
    g = sums[2 * out_rows:3 * out_rows]
    h = sums[3 * out_rows:4 * out_rows]
    k = sums[4 * out_rows:5 * out_rows]
    ef2 = 2.0 * (e * f)
    sq = e * e + f * f
    a1 = ef2 + c1s
    b1 = sq + c1s
    a2 = (2.0 * _NP) * k - ef2 + c2s
    b2 = _NP * (g + h) - sq + c2s
    s = (a1 * a2) / (b1 * b2)
    lane = jax.lax.broadcasted_iota(jnp.int32, s.shape, 1)
    return jnp.where(lane < _WO, s, 0.0)


def _ssim_kernel(p_ref, t_ref, b_ref, o_ref):
    C = p_ref.shape[1]

    p_all = p_ref[...]
    dr = jnp.max(p_all) - jnp.min(p_all)
    c1s = (_K1 * dr) ** 2 * (_NP * _NP)
    c2s = (_K2 * dr) ** 2 * (_NP * (_NP - 1))
    band = b_ref[...]

    def body(idx, acc):
        c = idx // _NSTRIP
        i = idx - c * _NSTRIP
        r0 = i * _STRIP
        p = p_ref[0, c, pl.ds(r0, _STRIP + _WIN - 1), :]
        t = t_ref[0, c, pl.ds(r0, _STRIP + _WIN - 1), :]
        return acc + _ssim_map(p, t, band, c1s, c2s, _STRIP)

    acc = jnp.zeros((_STRIP, _W), jnp.float32)
    acc = jax.lax.fori_loop(0, C * _NSTRIP, body, acc)

    def tail(c, acc):
        r0 = _NSTRIP * _STRIP
        p = p_ref[0, c, pl.ds(r0, _H - r0), :]
        t = t_ref[0, c, pl.ds(r0, _H - r0), :]
        s = _ssim_map(p, t, band, c1s, c2s, _REM)
        pad = jnp.zeros((_STRIP - _REM, _W), jnp.float32)
        return acc + jnp.concatenate([s, pad], axis=0)

    acc = jax.lax.fori_loop(0, C, tail, acc)

    mean_s = jnp.sum(acc) * (1.0 / (C * _HO * _WO))
    o_ref[...] = jnp.full(o_ref.shape, mean_s, jnp.float32)


def kernel(pred, target):
    B, C, H, W = pred.shape
    row = jax.lax.broadcasted_iota(jnp.int32, (W, W), 0)
    col = jax.lax.broadcasted_iota(jnp.int32, (W, W), 1)
    d = row - col
    band = jnp.where((d >= 0) & (d <= _WIN - 1) & (col < _WO), 1.0, 0.0)
    band = band.astype(jnp.bfloat16)
    per_sample = pl.pallas_call(
        _ssim_kernel,
        grid=(B,),
        in_specs=[
            pl.BlockSpec((1, C, H, W), lambda b: (b, 0, 0, 0)),
            pl.BlockSpec((1, C, H, W), lambda b: (b, 0, 0, 0)),
            pl.BlockSpec((W, W), lambda b: (0, 0)),
        ],
        out_specs=pl.BlockSpec((1, 1, 128), lambda b: (b, 0, 0)),
        out_shape=jax.ShapeDtypeStruct((B, 1, 128), jnp.float32),
        compiler_params=pltpu.CompilerParams(
            dimension_semantics=("parallel",),
            vmem_limit_bytes=56 * 1024 * 1024,
        ),
    )(pred, target, band)
    return 1.0 - jnp.mean(per_sample[:, 0, 0])
